# baseline (speedup 1.0000x reference)
import functools

import jax
import jax.numpy as jnp
from jax import lax
from jax.experimental import pallas as pl
from jax.experimental.pallas import tpu as pltpu
from jax.experimental.pallas import tpu_sc as plsc

LATENT = 32
_SQRT2 = 1.4142135623730951


def _gelu(x):
    return 0.5 * x * (1.0 + lax.erf(x / _SQRT2))


_NC = 2
_NS = 16
_NW = _NC * _NS
_CH = 128


_SZ = 1280
_K = _SZ // _CH
_C = 1


def _make_gather(E_pad, N_rows, D, N_pad=0, with_counts=False):
    per_w = E_pad // _NW
    n_sc = per_w // _SZ
    rows_t = N_pad // _NS if with_counts else 0
    mesh = plsc.VectorSubcoreMesh(core_axis_name="c", subcore_axis_name="s")

    out_type = [jax.ShapeDtypeStruct((E_pad, D), jnp.bfloat16)]
    scratch = [
        pltpu.VMEM((2, _SZ), jnp.int32),
        pltpu.VMEM((2, _SZ, D), jnp.bfloat16),
        pltpu.SemaphoreType.DMA,
        pltpu.SemaphoreType.DMA,
    ]
    if with_counts:
        out_type.append(jax.ShapeDtypeStruct((_NC, N_pad, 16), jnp.float32))
        scratch += [
            pltpu.VMEM((2, _K, _CH), jnp.int32),
            pltpu.VMEM((_CH, 16), jnp.float32),
            pltpu.VMEM_SHARED((N_pad, 16), jnp.float32),
            pltpu.SemaphoreType.DMA,
        ]

    def body_fn(x_hbm, idx_hbm, *refs):
        if with_counts:
            (dst2_hbm, out_hbm, cnt_hbm, idx_v, rows_v, semg, sems,
             didx_v, ones_v, cnt_sh, sem2) = refs
        else:
            (out_hbm, idx_v, rows_v, semg, sems) = refs
        c = lax.axis_index("c")
        s = lax.axis_index("s")
        base = (s * _NC + c) * per_w

        if with_counts:
            def zero_ones(i, carry):
                ones_v[i, pl.ds(0, 16)] = jnp.zeros((16,), jnp.float32)
                return carry
            lax.fori_loop(0, _CH, zero_ones, 0)

            def wipe_cnt(i, carry):
                pltpu.sync_copy(ones_v,
                                cnt_sh.at[pl.ds(s * rows_t + i * _CH, _CH)])
                return carry
            lax.fori_loop(0, rows_t // _CH, wipe_cnt, 0)

            def ones_body(i, carry):
                ones_v[i, pl.ds(0, 16)] = jnp.ones((16,), jnp.float32)
                return carry
            lax.fori_loop(0, _CH, ones_body, 0)
            plsc.subcore_barrier()

        g_prev = None
        cnt_copies = []
        pend = [None, None]
        for i in range(n_sc):
            b = i % 2
            off = base + i * _SZ
            pltpu.sync_copy(idx_hbm.at[pl.ds(off, _SZ)], idx_v.at[b])
            if with_counts:
                pltpu.sync_copy(dst2_hbm.at[pl.ds(off // _CH, _K)],
                                didx_v.at[b])
            if g_prev is not None:
                for cp in g_prev:
                    cp.wait()
                pend[1 - b] = pltpu.async_copy(
                    rows_v.at[1 - b], out_hbm.at[pl.ds(off - _SZ, _SZ)],
                    sems)
            if pend[b] is not None:
                pend[b].wait()
                pend[b] = None
            g_prev = [
                pltpu.async_copy(
                    x_hbm.at[idx_v.at[b, pl.ds(j * _CH, _CH)]],
                    rows_v.at[b, pl.ds(j * _CH, _CH)], semg)
                for j in range(_K)
            ]
            if with_counts:
                cnt_copies += [
                    pltpu.async_copy(ones_v, cnt_sh.at[didx_v.at[b, j]],
                                     sem2, add=True)
                    for j in range(_K)
                ]
        for cp in g_prev:
            cp.wait()
        pend[(n_sc - 1) % 2] = pltpu.async_copy(
            rows_v.at[(n_sc - 1) % 2],
            out_hbm.at[pl.ds(base + (n_sc - 1) * _SZ, _SZ)], sems)
        for cp in pend:
            if cp is not None:
                cp.wait()
        for cp in cnt_copies:
            cp.wait()

        if with_counts:
            plsc.subcore_barrier()
            pltpu.sync_copy(cnt_sh.at[pl.ds(s * rows_t, rows_t)],
                            cnt_hbm.at[c, pl.ds(s * rows_t, rows_t)])

    ot = tuple(out_type) if with_counts else out_type[0]
    return functools.partial(
        pl.kernel, mesh=mesh, out_type=ot,
        compiler_params=pltpu.CompilerParams(use_tc_tiling_on_sc=False),
        scratch_types=scratch)(body_fn)


def _make_scatter(E_pad, N_pad, D):
    per_w = E_pad // _NW
    n_sc = per_w // _SZ
    rows_t = N_pad // _NS
    mesh = plsc.VectorSubcoreMesh(core_axis_name="c", subcore_axis_name="s")

    @functools.partial(
        pl.kernel, mesh=mesh,
        out_type=jax.ShapeDtypeStruct((_NC, N_pad, D), jnp.float32),
        compiler_params=pltpu.CompilerParams(use_tc_tiling_on_sc=False),
        scratch_types=[
            pltpu.VMEM((2, _K, _CH), jnp.int32),
            pltpu.VMEM((2, _SZ, D), jnp.float32),
            pltpu.VMEM_SHARED((N_pad, D), jnp.float32),
            pltpu.SemaphoreType.DMA,
        ],
    )
    def body_fn(msg_hbm, idx2_hbm, out_hbm, idx_v, vals_v, acc_sh, sem):
        c = lax.axis_index("c")
        s = lax.axis_index("s")
        base = (s * _NC + c) * per_w

        def zero_body(i, carry):
            vals_v[0, i // (D // 16), pl.ds((i % (D // 16)) * 16, 16)] = (
                jnp.zeros((16,), jnp.float32))
            return carry
        lax.fori_loop(0, _CH * D // 16, zero_body, 0)

        def wipe(i, carry):
            pltpu.sync_copy(vals_v.at[0, pl.ds(0, _CH)],
                            acc_sh.at[pl.ds(s * rows_t + i * _CH, _CH)])
            return carry
        lax.fori_loop(0, rows_t // _CH, wipe, 0)

        plsc.subcore_barrier()

        a_prev = None
        for i in range(n_sc):
            b = i % 2
            off = base + i * _SZ
            pltpu.sync_copy(idx2_hbm.at[pl.ds(off // _CH, _K)],
                            idx_v.at[b])
            pltpu.sync_copy(msg_hbm.at[pl.ds(off, _SZ)], vals_v.at[b])
            if a_prev is not None:
                for cp in a_prev:
                    cp.wait()
            a_prev = [
                pltpu.async_copy(vals_v.at[b, pl.ds(j * _CH, _CH)],
                                 acc_sh.at[idx_v.at[b, j]], sem, add=True)
                for j in range(_K)
            ]
        for cp in a_prev:
            cp.wait()

        plsc.subcore_barrier()

        pltpu.sync_copy(acc_sh.at[pl.ds(s * rows_t, rows_t)],
                        out_hbm.at[c, pl.ds(s * rows_t, rows_t)])

    return body_fn


def _proj_body(x_ref, w1_ref, b1_ref, w2_ref, b2_ref, out_ref, outb_ref):
    h = _gelu(jnp.dot(x_ref[...], w1_ref[...],
                      preferred_element_type=jnp.float32) + b1_ref[...])
    x = _gelu(jnp.dot(h, w2_ref[...],
                      preferred_element_type=jnp.float32) + b2_ref[...])
    out_ref[...] = x
    outb_ref[...] = x.astype(jnp.bfloat16)


def _edge_body(ea_ref, xs_ref, w1_ref, b1_ref, w2b_ref, b2m_ref, selb_ref,
               msg_ref):
    h = _gelu(jnp.dot(ea_ref[...], w1_ref[...],
                      preferred_element_type=jnp.float32) + b1_ref[...])
    kerb = jnp.dot(h.astype(jnp.bfloat16), w2b_ref[...],
                   preferred_element_type=jnp.float32).astype(jnp.bfloat16)
    xs = xs_ref[...]
    xtb = jnp.concatenate([xs] * LATENT, axis=1)
    msg_ref[...] = (jnp.dot(kerb * xtb, selb_ref[...],
                            preferred_element_type=jnp.float32)
                    + jnp.dot(xs, b2m_ref[...].astype(jnp.bfloat16),
                              preferred_element_type=jnp.float32))


def _merge_agg(refs):
    p_refs, cnt_refs = refs[:_C], refs[_C:]
    agg = sum(p[i] for p in p_refs for i in range(_NC))
    deg = jnp.maximum(
        sum(cr[i, :, 0:1] for cr in cnt_refs for i in range(_NC)), 1.0)
    return agg / deg


def _update_body(x_ref, *refs, act):
    (w_ref, b_ref, out_ref, outb_ref) = refs[-4:]
    x = (jnp.dot(x_ref[...], w_ref[...],
                 preferred_element_type=jnp.float32) + b_ref[...]
         + _merge_agg(refs[:-4]))
    x = _gelu(x) if act else x
    out_ref[...] = x
    outb_ref[...] = x.astype(jnp.bfloat16)


def _update_dec_body(x_ref, *refs):
    (w_ref, b_ref, d1_ref, db1_ref, d2_ref, db2_ref, out_ref) = refs[-7:]
    x = (jnp.dot(x_ref[...], w_ref[...],
                 preferred_element_type=jnp.float32) + b_ref[...]
         + _merge_agg(refs[:-7]))
    h = _gelu(jnp.dot(x, d1_ref[...],
                      preferred_element_type=jnp.float32) + db1_ref[...])
    out_ref[...] = jnp.dot(h, d2_ref[...],
                           preferred_element_type=jnp.float32) + db2_ref[...]


def _full(shape):
    return pl.BlockSpec(shape, lambda i: (0,) * len(shape))


def _rows(bs, *trail):
    shape = (bs,) + trail
    return pl.BlockSpec(shape, lambda i: (i,) + (0,) * len(trail))


def kernel(nodes, grid, edge_index, edge_attr, batch_size, image_size,
           proj_W1, proj_b1, proj_W2, proj_b2,
           kern_W1, kern_b1, kern_W2, kern_b2,
           root_W, root_b,
           dec_W1, dec_b1, dec_W2, dec_b2):
    N, T_IN = nodes.shape
    E = edge_index.shape[1]
    EB = 8192
    NB = 1024
    E_pad = -(-E // 4096) * 4096
    N_pad = -(-N // (NB * 2)) * (NB * 2)

    src = jnp.pad(edge_index[0], (0, E_pad - E))
    dst2 = jnp.pad(edge_index[1], (0, E_pad - E),
                   constant_values=N).reshape(E_pad // _CH, _CH)
    ea = jnp.pad(edge_attr, ((0, E_pad - E), (0, 0)))
    x12 = jnp.pad(jnp.concatenate([nodes, grid], axis=1),
                  ((0, N_pad - N), (0, 0)))

    selb = (jnp.arange(LATENT * LATENT, dtype=jnp.int32)[:, None] // LATENT
            == jnp.arange(LATENT, dtype=jnp.int32)[None, :]
            ).astype(jnp.bfloat16)

    f32 = jnp.float32
    D_IN = T_IN + 2
    HID = proj_W1.shape[1]
    KER = kern_W1.shape[2]

    x, xb = pl.pallas_call(
        _proj_body,
        grid=(N_pad // NB,),
        in_specs=[_rows(NB, D_IN), _full((D_IN, HID)), _full((1, HID)),
                  _full((HID, LATENT)), _full((1, LATENT))],
        out_specs=[_rows(NB, LATENT), _rows(NB, LATENT)],
        out_shape=[jax.ShapeDtypeStruct((N_pad, LATENT), f32),
                   jax.ShapeDtypeStruct((N_pad, LATENT), jnp.bfloat16)],
    )(x12, proj_W1, proj_b1.reshape(1, -1), proj_W2, proj_b2.reshape(1, -1))

    gather0 = _make_gather(E_pad // _C, N_pad, LATENT, N_pad,
                           with_counts=True)
    gather1 = _make_gather(E_pad // _C, N_pad, LATENT)
    scatter_k = _make_scatter(E_pad // _C, N_pad, LATENT)

    edge_call = pl.pallas_call(
        _edge_body,
        grid=(E_pad // _C // EB,),
        in_specs=[_rows(EB, edge_attr.shape[1]), _rows(EB, LATENT),
                  _full((edge_attr.shape[1], KER)), _full((1, KER)),
                  _full((KER, LATENT * LATENT)),
                  _full((LATENT, LATENT)),
                  _full((LATENT * LATENT, LATENT))],
        out_specs=_rows(EB, LATENT),
        out_shape=jax.ShapeDtypeStruct((E_pad, LATENT), f32),
    )

    E_ch = E_pad // _C
    R2 = E_ch // _CH
    cnts = None
    depth = kern_W1.shape[0]
    for l in range(depth):
        parts = []
        if l == 0:
            cnts = []
        for k in range(_C):
            sl = slice(k * E_ch, (k + 1) * E_ch)
            if l == 0:
                xs, ck = gather0(xb, src[sl], dst2[k * R2:(k + 1) * R2])
                cnts.append(ck)
            else:
                xs = gather1(xb, src[sl])
            msg = edge_call(ea[sl], xs, kern_W1[l],
                            kern_b1[l].reshape(1, -1),
                            kern_W2[l].astype(jnp.bfloat16),
                            kern_b2[l].reshape(LATENT, LATENT).T, selb)
            parts.append(scatter_k(msg, dst2[k * R2:(k + 1) * R2]))

        upd_in = ([_rows(NB, LATENT)]
                  + [pl.BlockSpec((_NC, NB, LATENT), lambda i: (0, i, 0))
                     for _ in range(_C)]
                  + [pl.BlockSpec((_NC, NB, 16), lambda i: (0, i, 0))
                     for _ in range(_C)]
                  + [_full((LATENT, LATENT)), _full((1, LATENT))])
        upd_args = [x] + parts + cnts + [root_W[l],
                                         root_b[l].reshape(1, -1)]
        if l < depth - 1:
            x, xb = pl.pallas_call(
                functools.partial(_update_body, act=True),
                grid=(N_pad // NB,),
                in_specs=upd_in,
                out_specs=[_rows(NB, LATENT), _rows(NB, LATENT)],
                out_shape=[jax.ShapeDtypeStruct((N_pad, LATENT), f32),
                           jax.ShapeDtypeStruct((N_pad, LATENT),
                                                jnp.bfloat16)],
            )(*upd_args)
        else:
            out = pl.pallas_call(
                _update_dec_body,
                grid=(N_pad // NB,),
                in_specs=upd_in + [_full((LATENT, HID)), _full((1, HID)),
                                   _full((HID, 1)), _full((1, 1))],
                out_specs=_rows(NB, 1),
                out_shape=jax.ShapeDtypeStruct((N_pad, 1), f32),
            )(*(upd_args + [dec_W1, dec_b1.reshape(1, -1),
                            dec_W2, dec_b2.reshape(1, 1)]))

    return out[:N]


# speedup vs baseline: 1.0012x; 1.0012x over previous
import functools

import jax
import jax.numpy as jnp
from jax import lax
from jax.experimental import pallas as pl
from jax.experimental.pallas import tpu as pltpu
from jax.experimental.pallas import tpu_sc as plsc

LATENT = 32
_SQRT2 = 1.4142135623730951


def _gelu(x):
    return 0.5 * x * (1.0 + lax.erf(x / _SQRT2))


_NC = 2
_NS = 16
_NW = _NC * _NS
_CH = 128


_SZ = 1280
_K = _SZ // _CH
_C = 1


def _make_gather(E_pad, N_rows, D, N_pad=0, with_counts=False):
    per_w = E_pad // _NW
    n_sc = per_w // _SZ
    rows_t = N_pad // _NS if with_counts else 0
    mesh = plsc.VectorSubcoreMesh(core_axis_name="c", subcore_axis_name="s")

    out_type = [jax.ShapeDtypeStruct((E_pad, D), jnp.bfloat16)]
    scratch = [
        pltpu.VMEM((2, _SZ), jnp.int32),
        pltpu.VMEM((2, _SZ, D), jnp.bfloat16),
        pltpu.SemaphoreType.DMA,
        pltpu.SemaphoreType.DMA,
        pltpu.SemaphoreType.DMA,
    ]
    if with_counts:
        out_type.append(jax.ShapeDtypeStruct((_NC, N_pad, 16), jnp.float32))
        scratch += [
            pltpu.VMEM((2, _K, _CH), jnp.int32),
            pltpu.VMEM((_CH, 16), jnp.float32),
            pltpu.VMEM_SHARED((N_pad, 16), jnp.float32),
            pltpu.SemaphoreType.DMA,
        ]

    def body_fn(x_hbm, idx_hbm, *refs):
        if with_counts:
            (dst2_hbm, out_hbm, cnt_hbm, idx_v, rows_v, semg0, semg1, sems,
             didx_v, ones_v, cnt_sh, sem2) = refs
        else:
            (out_hbm, idx_v, rows_v, semg0, semg1, sems) = refs
        semg = (semg0, semg1)
        c = lax.axis_index("c")
        s = lax.axis_index("s")
        base = (s * _NC + c) * per_w

        if with_counts:
            def zero_ones(i, carry):
                ones_v[i, pl.ds(0, 16)] = jnp.zeros((16,), jnp.float32)
                return carry
            lax.fori_loop(0, _CH, zero_ones, 0)

            def wipe_cnt(i, carry):
                pltpu.sync_copy(ones_v,
                                cnt_sh.at[pl.ds(s * rows_t + i * _CH, _CH)])
                return carry
            lax.fori_loop(0, rows_t // _CH, wipe_cnt, 0)

            def ones_body(i, carry):
                ones_v[i, pl.ds(0, 16)] = jnp.ones((16,), jnp.float32)
                return carry
            lax.fori_loop(0, _CH, ones_body, 0)
            plsc.subcore_barrier()

        g_prev = None
        cnt_copies = []
        pend = [None, None]
        for i in range(n_sc):
            b = i % 2
            off = base + i * _SZ
            pltpu.sync_copy(idx_hbm.at[pl.ds(off, _SZ)], idx_v.at[b])
            if with_counts:
                pltpu.sync_copy(dst2_hbm.at[pl.ds(off // _CH, _K)],
                                didx_v.at[b])
            if pend[b] is not None:
                pend[b].wait()
                pend[b] = None
            g_new = [
                pltpu.async_copy(
                    x_hbm.at[idx_v.at[b, pl.ds(j * _CH, _CH)]],
                    rows_v.at[b, pl.ds(j * _CH, _CH)], semg[b])
                for j in range(_K)
            ]
            if with_counts:
                cnt_copies += [
                    pltpu.async_copy(ones_v, cnt_sh.at[didx_v.at[b, j]],
                                     sem2, add=True)
                    for j in range(_K)
                ]
            if g_prev is not None:
                for cp in g_prev:
                    cp.wait()
                pend[1 - b] = pltpu.async_copy(
                    rows_v.at[1 - b], out_hbm.at[pl.ds(off - _SZ, _SZ)],
                    sems)
            g_prev = g_new
        for cp in g_prev:
            cp.wait()
        pend[(n_sc - 1) % 2] = pltpu.async_copy(
            rows_v.at[(n_sc - 1) % 2],
            out_hbm.at[pl.ds(base + (n_sc - 1) * _SZ, _SZ)], sems)
        for cp in pend:
            if cp is not None:
                cp.wait()
        for cp in cnt_copies:
            cp.wait()

        if with_counts:
            plsc.subcore_barrier()
            pltpu.sync_copy(cnt_sh.at[pl.ds(s * rows_t, rows_t)],
                            cnt_hbm.at[c, pl.ds(s * rows_t, rows_t)])

    ot = tuple(out_type) if with_counts else out_type[0]
    return functools.partial(
        pl.kernel, mesh=mesh, out_type=ot,
        compiler_params=pltpu.CompilerParams(use_tc_tiling_on_sc=False),
        scratch_types=scratch)(body_fn)


def _make_scatter(E_pad, N_pad, D):
    per_w = E_pad // _NW
    n_sc = per_w // _SZ
    rows_t = N_pad // _NS
    mesh = plsc.VectorSubcoreMesh(core_axis_name="c", subcore_axis_name="s")

    @functools.partial(
        pl.kernel, mesh=mesh,
        out_type=jax.ShapeDtypeStruct((_NC, N_pad, D), jnp.float32),
        compiler_params=pltpu.CompilerParams(use_tc_tiling_on_sc=False),
        scratch_types=[
            pltpu.VMEM((2, _K, _CH), jnp.int32),
            pltpu.VMEM((2, _SZ, D), jnp.float32),
            pltpu.VMEM_SHARED((N_pad, D), jnp.float32),
            pltpu.SemaphoreType.DMA,
            pltpu.SemaphoreType.DMA,
        ],
    )
    def body_fn(msg_hbm, idx2_hbm, out_hbm, idx_v, vals_v, acc_sh,
                sem0, sem1):
        c = lax.axis_index("c")
        s = lax.axis_index("s")
        base = (s * _NC + c) * per_w

        def zero_body(i, carry):
            vals_v[0, i // (D // 16), pl.ds((i % (D // 16)) * 16, 16)] = (
                jnp.zeros((16,), jnp.float32))
            return carry
        lax.fori_loop(0, _CH * D // 16, zero_body, 0)

        def wipe(i, carry):
            pltpu.sync_copy(vals_v.at[0, pl.ds(0, _CH)],
                            acc_sh.at[pl.ds(s * rows_t + i * _CH, _CH)])
            return carry
        lax.fori_loop(0, rows_t // _CH, wipe, 0)

        plsc.subcore_barrier()

        sem = (sem0, sem1)
        a_prev = None
        for i in range(n_sc):
            b = i % 2
            off = base + i * _SZ
            pltpu.sync_copy(idx2_hbm.at[pl.ds(off // _CH, _K)],
                            idx_v.at[b])
            pltpu.sync_copy(msg_hbm.at[pl.ds(off, _SZ)], vals_v.at[b])
            a_new = [
                pltpu.async_copy(vals_v.at[b, pl.ds(j * _CH, _CH)],
                                 acc_sh.at[idx_v.at[b, j]], sem[b],
                                 add=True)
                for j in range(_K)
            ]
            if a_prev is not None:
                for cp in a_prev:
                    cp.wait()
            a_prev = a_new
        for cp in a_prev:
            cp.wait()

        plsc.subcore_barrier()

        pltpu.sync_copy(acc_sh.at[pl.ds(s * rows_t, rows_t)],
                        out_hbm.at[c, pl.ds(s * rows_t, rows_t)])

    return body_fn


def _proj_body(x_ref, w1_ref, b1_ref, w2_ref, b2_ref, out_ref, outb_ref):
    h = _gelu(jnp.dot(x_ref[...], w1_ref[...],
                      preferred_element_type=jnp.float32) + b1_ref[...])
    x = _gelu(jnp.dot(h, w2_ref[...],
                      preferred_element_type=jnp.float32) + b2_ref[...])
    out_ref[...] = x
    outb_ref[...] = x.astype(jnp.bfloat16)


def _edge_body(ea_ref, xs_ref, w1_ref, b1_ref, w2b_ref, b2m_ref, selb_ref,
               msg_ref):
    h = _gelu(jnp.dot(ea_ref[...], w1_ref[...],
                      preferred_element_type=jnp.float32) + b1_ref[...])
    kerb = jnp.dot(h.astype(jnp.bfloat16), w2b_ref[...],
                   preferred_element_type=jnp.float32).astype(jnp.bfloat16)
    xs = xs_ref[...]
    xtb = jnp.concatenate([xs] * LATENT, axis=1)
    msg_ref[...] = (jnp.dot(kerb * xtb, selb_ref[...],
                            preferred_element_type=jnp.float32)
                    + jnp.dot(xs, b2m_ref[...].astype(jnp.bfloat16),
                              preferred_element_type=jnp.float32))


def _merge_agg(refs):
    p_refs, cnt_refs = refs[:_C], refs[_C:]
    agg = sum(p[i] for p in p_refs for i in range(_NC))
    deg = jnp.maximum(
        sum(cr[i, :, 0:1] for cr in cnt_refs for i in range(_NC)), 1.0)
    return agg / deg


def _update_body(x_ref, *refs, act):
    (w_ref, b_ref, out_ref, outb_ref) = refs[-4:]
    x = (jnp.dot(x_ref[...], w_ref[...],
                 preferred_element_type=jnp.float32) + b_ref[...]
         + _merge_agg(refs[:-4]))
    x = _gelu(x) if act else x
    out_ref[...] = x
    outb_ref[...] = x.astype(jnp.bfloat16)


def _update_dec_body(x_ref, *refs):
    (w_ref, b_ref, d1_ref, db1_ref, d2_ref, db2_ref, out_ref) = refs[-7:]
    x = (jnp.dot(x_ref[...], w_ref[...],
                 preferred_element_type=jnp.float32) + b_ref[...]
         + _merge_agg(refs[:-7]))
    h = _gelu(jnp.dot(x, d1_ref[...],
                      preferred_element_type=jnp.float32) + db1_ref[...])
    out_ref[...] = jnp.dot(h, d2_ref[...],
                           preferred_element_type=jnp.float32) + db2_ref[...]


def _full(shape):
    return pl.BlockSpec(shape, lambda i: (0,) * len(shape))


def _rows(bs, *trail):
    shape = (bs,) + trail
    return pl.BlockSpec(shape, lambda i: (i,) + (0,) * len(trail))


def kernel(nodes, grid, edge_index, edge_attr, batch_size, image_size,
           proj_W1, proj_b1, proj_W2, proj_b2,
           kern_W1, kern_b1, kern_W2, kern_b2,
           root_W, root_b,
           dec_W1, dec_b1, dec_W2, dec_b2):
    N, T_IN = nodes.shape
    E = edge_index.shape[1]
    EB = 8192
    NB = 1024
    E_pad = -(-E // 4096) * 4096
    N_pad = -(-N // (NB * 2)) * (NB * 2)

    src = jnp.pad(edge_index[0], (0, E_pad - E))
    dst2 = jnp.pad(edge_index[1], (0, E_pad - E),
                   constant_values=N).reshape(E_pad // _CH, _CH)
    ea = jnp.pad(edge_attr, ((0, E_pad - E), (0, 0)))
    x12 = jnp.pad(jnp.concatenate([nodes, grid], axis=1),
                  ((0, N_pad - N), (0, 0)))

    selb = (jnp.arange(LATENT * LATENT, dtype=jnp.int32)[:, None] // LATENT
            == jnp.arange(LATENT, dtype=jnp.int32)[None, :]
            ).astype(jnp.bfloat16)

    f32 = jnp.float32
    D_IN = T_IN + 2
    HID = proj_W1.shape[1]
    KER = kern_W1.shape[2]

    x, xb = pl.pallas_call(
        _proj_body,
        grid=(N_pad // NB,),
        in_specs=[_rows(NB, D_IN), _full((D_IN, HID)), _full((1, HID)),
                  _full((HID, LATENT)), _full((1, LATENT))],
        out_specs=[_rows(NB, LATENT), _rows(NB, LATENT)],
        out_shape=[jax.ShapeDtypeStruct((N_pad, LATENT), f32),
                   jax.ShapeDtypeStruct((N_pad, LATENT), jnp.bfloat16)],
    )(x12, proj_W1, proj_b1.reshape(1, -1), proj_W2, proj_b2.reshape(1, -1))

    gather0 = _make_gather(E_pad // _C, N_pad, LATENT, N_pad,
                           with_counts=True)
    gather1 = _make_gather(E_pad // _C, N_pad, LATENT)
    scatter_k = _make_scatter(E_pad // _C, N_pad, LATENT)

    edge_call = pl.pallas_call(
        _edge_body,
        grid=(E_pad // _C // EB,),
        in_specs=[_rows(EB, edge_attr.shape[1]), _rows(EB, LATENT),
                  _full((edge_attr.shape[1], KER)), _full((1, KER)),
                  _full((KER, LATENT * LATENT)),
                  _full((LATENT, LATENT)),
                  _full((LATENT * LATENT, LATENT))],
        out_specs=_rows(EB, LATENT),
        out_shape=jax.ShapeDtypeStruct((E_pad, LATENT), f32),
    )

    E_ch = E_pad // _C
    R2 = E_ch // _CH
    cnts = None
    depth = kern_W1.shape[0]
    for l in range(depth):
        parts = []
        if l == 0:
            cnts = []
        for k in range(_C):
            sl = slice(k * E_ch, (k + 1) * E_ch)
            if l == 0:
                xs, ck = gather0(xb, src[sl], dst2[k * R2:(k + 1) * R2])
                cnts.append(ck)
            else:
                xs = gather1(xb, src[sl])
            msg = edge_call(ea[sl], xs, kern_W1[l],
                            kern_b1[l].reshape(1, -1),
                            kern_W2[l].astype(jnp.bfloat16),
                            kern_b2[l].reshape(LATENT, LATENT).T, selb)
            parts.append(scatter_k(msg, dst2[k * R2:(k + 1) * R2]))

        upd_in = ([_rows(NB, LATENT)]
                  + [pl.BlockSpec((_NC, NB, LATENT), lambda i: (0, i, 0))
                     for _ in range(_C)]
                  + [pl.BlockSpec((_NC, NB, 16), lambda i: (0, i, 0))
                     for _ in range(_C)]
                  + [_full((LATENT, LATENT)), _full((1, LATENT))])
        upd_args = [x] + parts + cnts + [root_W[l],
                                         root_b[l].reshape(1, -1)]
        if l < depth - 1:
            x, xb = pl.pallas_call(
                functools.partial(_update_body, act=True),
                grid=(N_pad // NB,),
                in_specs=upd_in,
                out_specs=[_rows(NB, LATENT), _rows(NB, LATENT)],
                out_shape=[jax.ShapeDtypeStruct((N_pad, LATENT), f32),
                           jax.ShapeDtypeStruct((N_pad, LATENT),
                                                jnp.bfloat16)],
            )(*upd_args)
        else:
            out = pl.pallas_call(
                _update_dec_body,
                grid=(N_pad // NB,),
                in_specs=upd_in + [_full((LATENT, HID)), _full((1, HID)),
                                   _full((HID, 1)), _full((1, 1))],
                out_specs=_rows(NB, 1),
                out_shape=jax.ShapeDtypeStruct((N_pad, 1), f32),
            )(*(upd_args + [dec_W1, dec_b1.reshape(1, -1),
                            dec_W2, dec_b2.reshape(1, 1)]))

    return out[:N]
